# Initial kernel scaffold; baseline (speedup 1.0000x reference)
import jax
import jax.numpy as jnp
from jax.experimental import pallas as pl


def kernel(x, W_patch, W_skip, W_desc, W_router, W_experts, W_fallback):
    raise NotImplementedError("write your pallas kernel here")


# speedup vs baseline: 1.3501x; 1.3501x over previous
import jax
import jax.numpy as jnp
from jax.experimental import pallas as pl
from jax.experimental.pallas import tpu as pltpu

_B = 8
_C_IN = 3
_H = 384
_W = 384
_P = 16
_N_TOK = (_H // _P) * (_W // _P)
_CP = _C_IN * _P * _P
_C = 384
_E = 4
_THR = 0.1


def _moe_step(patches_ref, Wps_ref, W_desc_ref, W_router_ref, W_all_ref,
              out_ref):
    both = jnp.dot(patches_ref[0], Wps_ref[...],
                   preferred_element_type=jnp.float32)
    feat = jnp.tanh(both[:, :_C])
    skip = both[:, _C:]

    mean = jnp.mean(feat, axis=0, keepdims=True)
    desc = jnp.dot(mean, W_desc_ref[...],
                   preferred_element_type=jnp.float32)
    logits = jnp.dot(desc, W_router_ref[...],
                     preferred_element_type=jnp.float32)
    lmax = jnp.max(logits)
    conf = 1.0 / jnp.sum(jnp.exp(logits - lmax))
    eid = jnp.argmax(logits[0]).astype(jnp.int32)
    idx = jnp.where(conf < _THR, jnp.int32(_E), eid)

    acc = jnp.dot(feat, W_all_ref[idx],
                  preferred_element_type=jnp.float32)
    out_ref[0] = jnp.maximum(acc, 0.0) + skip


def _patchify(x):
    b, c, h, w = x.shape
    x = x.reshape(b, c, h // _P, _P, w // _P, _P)
    x = x.transpose(0, 2, 4, 1, 3, 5)
    return x.reshape(b, (h // _P) * (w // _P), c * _P * _P)


@jax.jit
def kernel(x, W_patch, W_skip, W_desc, W_router, W_experts, W_fallback):
    patches = _patchify(x)
    Wps = jnp.concatenate([W_patch, W_skip], axis=1)
    W_all = jnp.concatenate([W_experts, W_fallback[None]], 0)

    grid_spec = pl.GridSpec(
        grid=(_B,),
        in_specs=[
            pl.BlockSpec((1, _N_TOK, _CP), lambda b: (b, 0, 0)),
            pl.BlockSpec((_CP, 2 * _C), lambda b: (0, 0)),
            pl.BlockSpec((_C, _C), lambda b: (0, 0)),
            pl.BlockSpec((_C, _E), lambda b: (0, 0)),
            pl.BlockSpec((_E + 1, _C, _C), lambda b: (0, 0, 0)),
        ],
        out_specs=pl.BlockSpec((1, _N_TOK, _C), lambda b: (b, 0, 0)),
    )
    return pl.pallas_call(
        _moe_step,
        grid_spec=grid_spec,
        out_shape=jax.ShapeDtypeStruct((_B, _N_TOK, _C), jnp.float32),
    )(patches, Wps, W_desc, W_router, W_all)


# speedup vs baseline: 3.3539x; 2.4842x over previous
import jax
import jax.numpy as jnp
from jax.experimental import pallas as pl
from jax.experimental.pallas import tpu as pltpu

_B = 8
_C_IN = 3
_H = 384
_W = 384
_P = 16
_N_TOK = (_H // _P) * (_W // _P)
_CP = _C_IN * _P * _P
_C = 384
_E = 4
_THR = 0.1


def _moe_step(x_ref, Wps_ref, W_desc_ref, W_router_ref, W_all_ref,
              out_ref):
    v = x_ref[0].reshape(_C_IN, _H // _P, _P, _W // _P, _P)
    patches = v.transpose(1, 3, 0, 2, 4).reshape(_N_TOK, _CP)
    both = jnp.dot(patches, Wps_ref[...],
                   preferred_element_type=jnp.float32)
    feat = jnp.tanh(both[:, :_C])
    skip = both[:, _C:]

    mean = jnp.mean(feat, axis=0, keepdims=True)
    desc = jnp.dot(mean, W_desc_ref[...],
                   preferred_element_type=jnp.float32)
    logits = jnp.dot(desc, W_router_ref[...],
                     preferred_element_type=jnp.float32)
    lmax = jnp.max(logits)
    conf = 1.0 / jnp.sum(jnp.exp(logits - lmax))
    eid = jnp.argmax(logits[0]).astype(jnp.int32)
    idx = jnp.where(conf < _THR, jnp.int32(_E), eid)

    acc = jnp.dot(feat, W_all_ref[idx],
                  preferred_element_type=jnp.float32)
    out_ref[0] = jnp.maximum(acc, 0.0) + skip


def _patchify(x):
    b, c, h, w = x.shape
    x = x.reshape(b, c, h // _P, _P, w // _P, _P)
    x = x.transpose(0, 2, 4, 1, 3, 5)
    return x.reshape(b, (h // _P) * (w // _P), c * _P * _P)


@jax.jit
def kernel(x, W_patch, W_skip, W_desc, W_router, W_experts, W_fallback):
    Wps = jnp.concatenate([W_patch, W_skip], axis=1)
    W_all = jnp.concatenate([W_experts, W_fallback[None]], 0)

    grid_spec = pl.GridSpec(
        grid=(_B,),
        in_specs=[
            pl.BlockSpec((1, _C_IN, _H, _W), lambda b: (b, 0, 0, 0)),
            pl.BlockSpec((_CP, 2 * _C), lambda b: (0, 0)),
            pl.BlockSpec((_C, _C), lambda b: (0, 0)),
            pl.BlockSpec((_C, _E), lambda b: (0, 0)),
            pl.BlockSpec((_E + 1, _C, _C), lambda b: (0, 0, 0)),
        ],
        out_specs=pl.BlockSpec((1, _N_TOK, _C), lambda b: (b, 0, 0)),
    )
    return pl.pallas_call(
        _moe_step,
        grid_spec=grid_spec,
        out_shape=jax.ShapeDtypeStruct((_B, _N_TOK, _C), jnp.float32),
    )(x, Wps, W_desc, W_router, W_all)
